# baseline (speedup 1.0000x reference)
import functools

import jax
import jax.numpy as jnp
from jax import lax
from jax.experimental import pallas as pl
from jax.experimental.pallas import tpu as pltpu
from jax.experimental.pallas import tpu_sc as plsc

_AVG_NEIGH = 16.0
_NEL = 10

_NC = 2
_NS = 16
_LANES = 16

_N = 10000
_E = 160000
_C = 128
_NCHUNK = 4
_CCH = 32
_B = 80
_EPT = _E // _NS
_NBLK = _EPT // _B
_NPAD = 10112
_NPT = _NPAD // _NS


def _matmul_body(x_ref, w_ref, o_ref):
    o_ref[...] = jnp.dot(x_ref[...], w_ref[...],
                         preferred_element_type=jnp.float32)


def _linear_up(node_feats, w):
    n, c = node_feats.shape
    bn = 1000
    return pl.pallas_call(
        _matmul_body,
        grid=(n // bn,),
        in_specs=[
            pl.BlockSpec((bn, c), lambda i: (i, 0)),
            pl.BlockSpec((c, c), lambda i: (0, 0)),
        ],
        out_specs=pl.BlockSpec((bn, c), lambda i: (i, 0)),
        out_shape=jax.ShapeDtypeStruct((n, c), jnp.float32),
    )(node_feats, w)


def _sc_message_body(h, efT, attrs, rcv, snd, zeros_hbm, out,
                     gidx0, gidx1, rcv0, rcv1, hrows0, hrows1, ef_v,
                     attr_v, out_v, scat_v, msg_sh,
                     semi0, semi1, semp0, semp1, sem_s):
    core = lax.axis_index("c")
    sub = lax.axis_index("s")
    gidx_v = (gidx0, gidx1)
    rcv_v = (rcv0, rcv1)
    hrows_v = (hrows0, hrows1)
    sem_i = (semi0, semi1)
    sem_p = (semp0, semp1)

    for cc in range(_NCHUNK // _NC):
        chunk = _NC * cc + core

        pltpu.sync_copy(zeros_hbm, msg_sh.at[pl.ds(sub * _NPT, _NPT), :])
        plsc.subcore_barrier()

        def _issue_idx(p, b):
            e0 = sub * _EPT + b * _B
            pltpu.async_copy(rcv.at[pl.ds(e0, _B)], rcv_v[p], sem_i[p])
            pltpu.async_copy(snd.at[pl.ds(e0, _B)], gidx_v[p], sem_i[p])

        def _wait_idx(p, b):
            e0 = sub * _EPT + b * _B
            pltpu.make_async_copy(rcv.at[pl.ds(e0, _B)], rcv_v[p],
                                  sem_i[p]).wait()
            pltpu.make_async_copy(snd.at[pl.ds(e0, _B)], gidx_v[p],
                                  sem_i[p]).wait()

        def _issue_payload(p, b):
            pltpu.async_copy(h.at[gidx_v[p]], hrows_v[p], sem_p[p])

        def _wait_payload(p, b):
            pltpu.make_async_copy(h.at[gidx_v[p]], hrows_v[p],
                                  sem_p[p]).wait()

        def _compute_scatter(p, b, first=False):
            e0 = sub * _EPT + b * _B
            pltpu.sync_copy(efT.at[pl.ds(chunk * _E + e0, _B), :], ef_v)
            pltpu.sync_copy(attrs.at[pl.ds(e0 * 4, _B * 4)],
                            attr_v.at[pl.ds(0, _B * 4)])
            if not first:
                pltpu.make_async_copy(out_v, msg_sh.at[scat_v],
                                      sem_s).wait()
            for j in range(_B // _LANES):
                scat_v[pl.ds(j * _LANES, _LANES)] = (
                    rcv_v[p][pl.ds(j * _LANES, _LANES)])

            def _edge(i, _):
                h0 = hrows_v[p][i, pl.ds(chunk * _CCH, _LANES)]
                h1 = hrows_v[p][i, pl.ds(chunk * _CCH + _LANES, _LANES)]
                e00 = ef_v[i, pl.ds(0, _LANES)]
                e01 = ef_v[i, pl.ds(_LANES, _LANES)]
                e10 = ef_v[i, pl.ds(2 * _LANES, _LANES)]
                e11 = ef_v[i, pl.ds(3 * _LANES, _LANES)]
                t00 = e00 * h0
                t01 = e01 * h1
                t10 = e10 * h0
                t11 = e11 * h1
                av = attr_v[pl.ds(i * 4, _LANES)]
                for m in range(4):
                    a = av[m]
                    u0, u1 = (t00, t01) if m == 0 else (t10, t11)
                    out_v[i, pl.ds(m * 2 * _LANES, _LANES)] = u0 * a
                    out_v[i, pl.ds((m * 2 + 1) * _LANES, _LANES)] = u1 * a
                return 0
            lax.fori_loop(0, _B, _edge, 0, unroll=8)
            pltpu.async_copy(out_v, msg_sh.at[scat_v], sem_s, add=True)

        def _phase(p, b, do_compute, issue_next, first=False):
            q = 1 - p
            _wait_idx(p, b)
            _issue_payload(p, b)
            if do_compute:
                _wait_payload(q, b - 1)
                _compute_scatter(q, b - 1, first=first)
            if issue_next:
                _issue_idx(q, b + 1)

        _issue_idx(0, 0)
        _phase(0, 0, do_compute=False, issue_next=True)
        _phase(1, 1, do_compute=True, issue_next=True, first=True)
        _phase(0, 2, do_compute=True, issue_next=True)

        def _pair(i, _):
            b = 2 * i + 1
            _phase(1, b, do_compute=True, issue_next=True)
            _phase(0, b + 1, do_compute=True, issue_next=True)
            return 0
        lax.fori_loop(1, (_NBLK - 3) // 2, _pair, 0)

        _phase(1, _NBLK - 2, do_compute=True, issue_next=True)
        _phase(0, _NBLK - 1, do_compute=True, issue_next=False)
        _wait_payload(0, _NBLK - 1)
        _compute_scatter(0, _NBLK - 1)
        pltpu.make_async_copy(out_v, msg_sh.at[scat_v], sem_s).wait()

        plsc.subcore_barrier()

        pltpu.sync_copy(
            msg_sh.at[pl.ds(sub * _NPT, _NPT), :],
            out.at[pl.ds(chunk * _NPAD + sub * _NPT, _NPT), :])


def _sc_message(h, efT, edge_attrs, rcv, snd, zeros_hbm):
    mesh = plsc.VectorSubcoreMesh(core_axis_name="c", subcore_axis_name="s",
                                  num_cores=_NC, num_subcores=_NS)
    f = pl.kernel(
        _sc_message_body,
        out_type=jax.ShapeDtypeStruct((_NCHUNK * _NPAD, _NCHUNK * _CCH),
                                      jnp.float32),
        mesh=mesh,
        scratch_types=[
            pltpu.VMEM((_B,), jnp.int32),
            pltpu.VMEM((_B,), jnp.int32),
            pltpu.VMEM((_B,), jnp.int32),
            pltpu.VMEM((_B,), jnp.int32),
            pltpu.VMEM((_B, _C), jnp.float32),
            pltpu.VMEM((_B, _C), jnp.float32),
            pltpu.VMEM((_B, 2 * _CCH), jnp.float32),
            pltpu.VMEM((_B * 4 + _LANES,), jnp.float32),
            pltpu.VMEM((_B, 4 * _CCH), jnp.float32),
            pltpu.VMEM((_B,), jnp.int32),
            pltpu.VMEM_SHARED((_NPAD, 4 * _CCH), jnp.float32),
            pltpu.SemaphoreType.DMA,
            pltpu.SemaphoreType.DMA,
            pltpu.SemaphoreType.DMA,
            pltpu.SemaphoreType.DMA,
            pltpu.SemaphoreType.DMA,
        ],
    )
    return f(h, efT, edge_attrs, rcv, snd, zeros_hbm)


def _mix_body(x0_ref, x1_ref, attrs_ref, wc_ref, o_ref):
    x0 = x0_ref[...]
    x1 = x1_ref[...]
    bn = x0.shape[0]
    a = attrs_ref[...]
    mx = jnp.max(a, axis=1, keepdims=True)
    li = lax.broadcasted_iota(jnp.int32, a.shape, 1)
    cand = jnp.where(a == mx, li, a.shape[1])
    fi = jnp.min(cand, axis=1, keepdims=True)
    oh = (li == fi).astype(jnp.float32)
    x0b = x0.astype(jnp.bfloat16)
    x1b = x1.astype(jnp.bfloat16)
    oh3 = jnp.broadcast_to(oh[:, None, :], (bn, 3, 128)).reshape(3 * bn, 128)
    acc0 = jnp.zeros((bn, _C), jnp.float32)
    acc1 = jnp.zeros((3 * bn, _C), jnp.float32)
    for e in range(_NEL):
        y0 = jnp.dot(x0b, wc_ref[e, 0], preferred_element_type=jnp.float32)
        y1 = jnp.dot(x1b, wc_ref[e, 1], preferred_element_type=jnp.float32)
        acc0 += y0 * oh[:, e][:, None]
        acc1 += y1 * oh3[:, e][:, None]
    o_ref[:, 0, :] = acc0
    o_ref[:, 1:, :] = acc1.reshape(bn, 3, _C)


def _mix(x0, x1, attrs_pad, wc_bf):
    n = _N
    c = _C
    bn = 1000
    return pl.pallas_call(
        _mix_body,
        grid=(n // bn,),
        in_specs=[
            pl.BlockSpec((bn, c), lambda i: (i, 0)),
            pl.BlockSpec((3 * bn, c), lambda i: (i, 0)),
            pl.BlockSpec((bn, 128), lambda i: (i, 0)),
            pl.BlockSpec((_NEL, 2, c, c), lambda i: (0, 0, 0, 0)),
        ],
        out_specs=pl.BlockSpec((bn, 4, c), lambda i: (i, 0, 0)),
        out_shape=jax.ShapeDtypeStruct((n, 4, c), jnp.float32),
    )(x0, x1, attrs_pad, wc_bf)


def kernel(node_attrs, node_feats, edge_attrs, edge_feats, W_up, W_lin,
           W_skip, edge_index):
    n, c = node_feats.shape
    e_cnt = edge_attrs.shape[0]

    h = _linear_up(node_feats, W_up)

    efT = edge_feats.reshape(e_cnt, 2, _NCHUNK, _CCH).transpose(
        2, 0, 1, 3).reshape(_NCHUNK * e_cnt, 2 * _CCH)

    zeros_hbm = jnp.zeros((_NPT, 4 * _CCH), jnp.float32)
    msg = _sc_message(h, efT, edge_attrs.reshape(-1), edge_index[0],
                      edge_index[1], zeros_hbm)
    msg4 = msg.reshape(_NCHUNK, _NPAD, 4, _CCH)
    x0 = msg4[:, :n, 0, :].transpose(1, 0, 2).reshape(n, c)
    x1 = msg4[:, :n, 1:, :].transpose(1, 2, 0, 3).reshape(3 * n, c)

    wc_bf = (jnp.einsum('lcd,eldf->elcf', W_lin, W_skip)
             / _AVG_NEIGH).astype(jnp.bfloat16)
    attrs_pad = jnp.pad(node_attrs, ((0, 0), (0, 128 - node_attrs.shape[1])))
    return _mix(x0, x1, attrs_pad, wc_bf)


# speedup vs baseline: 1.0051x; 1.0051x over previous
import functools

import jax
import jax.numpy as jnp
from jax import lax
from jax.experimental import pallas as pl
from jax.experimental.pallas import tpu as pltpu
from jax.experimental.pallas import tpu_sc as plsc

_AVG_NEIGH = 16.0
_NEL = 10

_NC = 2
_NS = 16
_LANES = 16

_N = 10000
_E = 160000
_C = 128
_NCHUNK = 4
_CCH = 32
_B = 80
_EPT = _E // _NS
_NBLK = _EPT // _B
_NPAD = 10112
_NPT = _NPAD // _NS


def _matmul_body(x_ref, w_ref, o_ref):
    o_ref[...] = jnp.dot(x_ref[...], w_ref[...],
                         preferred_element_type=jnp.float32)


def _linear_up(node_feats, w):
    n, c = node_feats.shape
    bn = 1000
    return pl.pallas_call(
        _matmul_body,
        grid=(n // bn,),
        in_specs=[
            pl.BlockSpec((bn, c), lambda i: (i, 0)),
            pl.BlockSpec((c, c), lambda i: (0, 0)),
        ],
        out_specs=pl.BlockSpec((bn, c), lambda i: (i, 0)),
        out_shape=jax.ShapeDtypeStruct((n, c), jnp.float32),
    )(node_feats, w)


def _sc_message_body(h, efT, attrs, rcv, snd, zeros_hbm, out,
                     gidx0, gidx1, rcv0, rcv1, hrows0, hrows1, ef_v,
                     attr_v, out_v, scat_v, msg_sh,
                     semi0, semi1, semp0, semp1, sem_s):
    core = lax.axis_index("c")
    sub = lax.axis_index("s")
    gidx_v = (gidx0, gidx1)
    rcv_v = (rcv0, rcv1)
    hrows_v = (hrows0, hrows1)
    sem_i = (semi0, semi1)
    sem_p = (semp0, semp1)

    for cc in range(_NCHUNK // _NC):
        chunk = _NC * cc + core

        pltpu.sync_copy(zeros_hbm, msg_sh.at[pl.ds(sub * _NPT, _NPT), :])
        plsc.subcore_barrier()

        def _issue_idx(p, b):
            e0 = sub * _EPT + b * _B
            pltpu.async_copy(rcv.at[pl.ds(e0, _B)], rcv_v[p], sem_i[p])
            pltpu.async_copy(snd.at[pl.ds(e0, _B)], gidx_v[p], sem_i[p])

        def _wait_idx(p, b):
            e0 = sub * _EPT + b * _B
            pltpu.make_async_copy(rcv.at[pl.ds(e0, _B)], rcv_v[p],
                                  sem_i[p]).wait()
            pltpu.make_async_copy(snd.at[pl.ds(e0, _B)], gidx_v[p],
                                  sem_i[p]).wait()

        def _issue_payload(p, b):
            pltpu.async_copy(h.at[gidx_v[p]], hrows_v[p], sem_p[p])

        def _wait_payload(p, b):
            pltpu.make_async_copy(h.at[gidx_v[p]], hrows_v[p],
                                  sem_p[p]).wait()

        def _compute_scatter(p, b, first=False):
            e0 = sub * _EPT + b * _B
            pltpu.sync_copy(efT.at[pl.ds(chunk * _E + e0, _B), :], ef_v)
            pltpu.sync_copy(attrs.at[pl.ds(e0 * 4, _B * 4)],
                            attr_v.at[pl.ds(0, _B * 4)])
            if not first:
                pltpu.make_async_copy(out_v, msg_sh.at[scat_v],
                                      sem_s).wait()
            for j in range(_B // _LANES):
                scat_v[pl.ds(j * _LANES, _LANES)] = (
                    rcv_v[p][pl.ds(j * _LANES, _LANES)])

            def _edge(i, _):
                h0 = hrows_v[p][i, pl.ds(chunk * _CCH, _LANES)]
                h1 = hrows_v[p][i, pl.ds(chunk * _CCH + _LANES, _LANES)]
                e00 = ef_v[i, pl.ds(0, _LANES)]
                e01 = ef_v[i, pl.ds(_LANES, _LANES)]
                e10 = ef_v[i, pl.ds(2 * _LANES, _LANES)]
                e11 = ef_v[i, pl.ds(3 * _LANES, _LANES)]
                t00 = e00 * h0
                t01 = e01 * h1
                t10 = e10 * h0
                t11 = e11 * h1
                av = attr_v[pl.ds(i * 4, _LANES)]
                for m in range(4):
                    a = av[m]
                    u0, u1 = (t00, t01) if m == 0 else (t10, t11)
                    out_v[i, pl.ds(m * 2 * _LANES, _LANES)] = u0 * a
                    out_v[i, pl.ds((m * 2 + 1) * _LANES, _LANES)] = u1 * a
                return 0
            lax.fori_loop(0, _B, _edge, 0, unroll=4)
            pltpu.async_copy(out_v, msg_sh.at[scat_v], sem_s, add=True)

        def _phase(p, b, do_compute, issue_next, first=False):
            q = 1 - p
            _wait_idx(p, b)
            _issue_payload(p, b)
            if do_compute:
                _wait_payload(q, b - 1)
                _compute_scatter(q, b - 1, first=first)
            if issue_next:
                _issue_idx(q, b + 1)

        _issue_idx(0, 0)
        _phase(0, 0, do_compute=False, issue_next=True)
        _phase(1, 1, do_compute=True, issue_next=True, first=True)
        _phase(0, 2, do_compute=True, issue_next=True)

        def _pair(i, _):
            b = 2 * i + 1
            _phase(1, b, do_compute=True, issue_next=True)
            _phase(0, b + 1, do_compute=True, issue_next=True)
            return 0
        lax.fori_loop(1, (_NBLK - 3) // 2, _pair, 0)

        _phase(1, _NBLK - 2, do_compute=True, issue_next=True)
        _phase(0, _NBLK - 1, do_compute=True, issue_next=False)
        _wait_payload(0, _NBLK - 1)
        _compute_scatter(0, _NBLK - 1)
        pltpu.make_async_copy(out_v, msg_sh.at[scat_v], sem_s).wait()

        plsc.subcore_barrier()

        pltpu.sync_copy(
            msg_sh.at[pl.ds(sub * _NPT, _NPT), :],
            out.at[pl.ds(chunk * _NPAD + sub * _NPT, _NPT), :])


def _sc_message(h, efT, edge_attrs, rcv, snd, zeros_hbm):
    mesh = plsc.VectorSubcoreMesh(core_axis_name="c", subcore_axis_name="s",
                                  num_cores=_NC, num_subcores=_NS)
    f = pl.kernel(
        _sc_message_body,
        out_type=jax.ShapeDtypeStruct((_NCHUNK * _NPAD, _NCHUNK * _CCH),
                                      jnp.float32),
        mesh=mesh,
        scratch_types=[
            pltpu.VMEM((_B,), jnp.int32),
            pltpu.VMEM((_B,), jnp.int32),
            pltpu.VMEM((_B,), jnp.int32),
            pltpu.VMEM((_B,), jnp.int32),
            pltpu.VMEM((_B, _C), jnp.float32),
            pltpu.VMEM((_B, _C), jnp.float32),
            pltpu.VMEM((_B, 2 * _CCH), jnp.float32),
            pltpu.VMEM((_B * 4 + _LANES,), jnp.float32),
            pltpu.VMEM((_B, 4 * _CCH), jnp.float32),
            pltpu.VMEM((_B,), jnp.int32),
            pltpu.VMEM_SHARED((_NPAD, 4 * _CCH), jnp.float32),
            pltpu.SemaphoreType.DMA,
            pltpu.SemaphoreType.DMA,
            pltpu.SemaphoreType.DMA,
            pltpu.SemaphoreType.DMA,
            pltpu.SemaphoreType.DMA,
        ],
    )
    return f(h, efT, edge_attrs, rcv, snd, zeros_hbm)


def _mix_body(x0_ref, x1_ref, attrs_ref, wc_ref, o_ref):
    x0 = x0_ref[...]
    x1 = x1_ref[...]
    bn = x0.shape[0]
    a = attrs_ref[...]
    mx = jnp.max(a, axis=1, keepdims=True)
    li = lax.broadcasted_iota(jnp.int32, a.shape, 1)
    cand = jnp.where(a == mx, li, a.shape[1])
    fi = jnp.min(cand, axis=1, keepdims=True)
    oh = (li == fi).astype(jnp.float32)
    x0b = x0.astype(jnp.bfloat16)
    x1b = x1.astype(jnp.bfloat16)
    acc0 = jnp.zeros((bn, _C), jnp.float32)
    acc1 = jnp.zeros((3 * bn, _C), jnp.float32)
    for e in range(_NEL):
        y0 = jnp.dot(x0b, wc_ref[e, 0], preferred_element_type=jnp.float32)
        y1 = jnp.dot(x1b, wc_ref[e, 1], preferred_element_type=jnp.float32)
        m0 = oh[:, e][:, None]
        m1 = jnp.broadcast_to(m0[:, None, :], (bn, 3, 1)).reshape(3 * bn, 1)
        acc0 += y0 * m0
        acc1 += y1 * m1
    o_ref[:, 0, :] = acc0
    o_ref[:, 1:, :] = acc1.reshape(bn, 3, _C)


def _mix(x0, x1, attrs_pad, wc_bf):
    n = _N
    c = _C
    bn = 1000
    return pl.pallas_call(
        _mix_body,
        grid=(n // bn,),
        in_specs=[
            pl.BlockSpec((bn, c), lambda i: (i, 0)),
            pl.BlockSpec((3 * bn, c), lambda i: (i, 0)),
            pl.BlockSpec((bn, 128), lambda i: (i, 0)),
            pl.BlockSpec((_NEL, 2, c, c), lambda i: (0, 0, 0, 0)),
        ],
        out_specs=pl.BlockSpec((bn, 4, c), lambda i: (i, 0, 0)),
        out_shape=jax.ShapeDtypeStruct((n, 4, c), jnp.float32),
    )(x0, x1, attrs_pad, wc_bf)


def kernel(node_attrs, node_feats, edge_attrs, edge_feats, W_up, W_lin,
           W_skip, edge_index):
    n, c = node_feats.shape
    e_cnt = edge_attrs.shape[0]

    h = _linear_up(node_feats, W_up)

    efT = edge_feats.reshape(e_cnt, 2, _NCHUNK, _CCH).transpose(
        2, 0, 1, 3).reshape(_NCHUNK * e_cnt, 2 * _CCH)

    zeros_hbm = jnp.zeros((_NPT, 4 * _CCH), jnp.float32)
    msg = _sc_message(h, efT, edge_attrs.reshape(-1), edge_index[0],
                      edge_index[1], zeros_hbm)
    msg4 = msg.reshape(_NCHUNK, _NPAD, 4, _CCH)
    x0 = msg4[:, :n, 0, :].transpose(1, 0, 2).reshape(n, c)
    x1 = msg4[:, :n, 1:, :].transpose(1, 2, 0, 3).reshape(3 * n, c)

    wc_bf = (jnp.einsum('lcd,eldf->elcf', W_lin, W_skip)
             / _AVG_NEIGH).astype(jnp.bfloat16)
    attrs_pad = jnp.pad(node_attrs, ((0, 0), (0, 128 - node_attrs.shape[1])))
    return _mix(x0, x1, attrs_pad, wc_bf)
